# baseline (speedup 1.0000x reference)
import functools

import jax
import jax.numpy as jnp
from jax import lax
from jax.experimental import pallas as pl
from jax.experimental.pallas import tpu as pltpu
from jax.experimental.pallas import tpu_sc as plsc

B, S, D = 4, 8192, 1024
K = max(1, int(0.5 * S))
ONE_BITS = 0x3F800000

TC_S = 4096
SC_S = S - TC_S
BLK = 512
NBLK = TC_S // BLK

NW = 32
TOK_PER_W = B * SC_S // NW
CHUNKS_PER_B = NW // B
TILE_T = 32
NT = TOK_PER_W // TILE_T
NL = D // 16

_SPLIT_C = jnp.float32(65537.0)


def _bf16_round(x):
    t = x * _SPLIT_C
    return t - (t - x)


def _tc_matvec_body(hs_ref, w_ref, b_ref, weights_ref):
    i = pl.program_id(0)
    hs = hs_ref[...]
    w = w_ref[...]
    logits = jnp.dot(hs.reshape(B * BLK, D), w.reshape(D, 1),
                     preferred_element_type=jnp.float32)
    logits = logits.reshape(B, BLK) + b_ref[0, 0]
    weights_ref[...] = jax.nn.sigmoid(logits)


def _tc_weights(hidden_states, W, b):
    return pl.pallas_call(
        _tc_matvec_body,
        grid=(NBLK,),
        in_specs=[
            pl.BlockSpec((B, BLK, D), lambda i: (0, i, 0)),
            pl.BlockSpec((1, D), lambda i: (0, 0)),
            pl.BlockSpec((1, 1), lambda i: (0, 0)),
        ],
        out_specs=pl.BlockSpec((B, BLK), lambda i: (0, i)),
        out_shape=jax.ShapeDtypeStruct((B, TC_S), jnp.float32),
    )(hidden_states, W, b.reshape(1, 1))


@functools.partial(
    pl.kernel,
    mesh=plsc.VectorSubcoreMesh(core_axis_name="c", subcore_axis_name="s"),
    compiler_params=pltpu.CompilerParams(use_tc_tiling_on_sc=False),
    out_type=jax.ShapeDtypeStruct((B, SC_S), jnp.float32),
    scratch_types=[
        pltpu.VMEM((D,), jnp.float32),
        pltpu.VMEM((16,), jnp.float32),
        pltpu.VMEM((TILE_T * D,), jnp.float32),
        pltpu.VMEM((TILE_T * D,), jnp.float32),
        pltpu.VMEM((TOK_PER_W,), jnp.float32),
        pltpu.SemaphoreType.DMA,
        pltpu.SemaphoreType.DMA,
    ],
)
def _sc_weights(hs_hbm, w_hbm, b_hbm, out_hbm,
                wv, bv, buf0, buf1, outv, sem0, sem1):
    cid = lax.axis_index("c")
    sid = lax.axis_index("s")
    wid = sid * 2 + cid
    batch = wid // CHUNKS_PER_B
    chunk = wid % CHUNKS_PER_B
    base = TC_S + chunk * TOK_PER_W

    pltpu.sync_copy(w_hbm, wv)
    pltpu.sync_copy(b_hbm, bv)
    bufs = (buf0, buf1)
    sems = (sem0, sem1)

    lanes = lax.iota(jnp.int32, 16)
    rot_idx = [(lanes + sh) & 15 for sh in (8, 4, 2, 1)]
    dn = lax.GatherDimensionNumbers(
        offset_dims=(), collapsed_slice_dims=(0,), start_index_map=(0,))

    def _rot(v, idx):
        return lax.gather(v, idx[:, None], dn, (1,),
                          mode=lax.GatherScatterMode.PROMISE_IN_BOUNDS)

    zero = jnp.zeros((16,), jnp.float32)
    bvec = bv[...]

    copies = [None, None]
    copies[0] = pltpu.async_copy(
        hs_hbm.at[batch, pl.ds(base * D, TILE_T * D)], buf0, sems[0])
    for t in range(NT):
        if t + 1 < NT:
            copies[(t + 1) % 2] = pltpu.async_copy(
                hs_hbm.at[batch, pl.ds((base + (t + 1) * TILE_T) * D, TILE_T * D)],
                bufs[(t + 1) % 2], sems[(t + 1) % 2])
        copies[t % 2].wait()
        btile = bufs[t % 2]

        for g in range(TILE_T // 16):
            goff = (t * TILE_T + g * 16) - (t * TILE_T)
            row0 = g * 16 * D

            def _jstep(j, accs):
                wchunk = _bf16_round(wv[pl.ds(j * 16, 16)])
                return tuple(
                    acc + _bf16_round(
                        btile[pl.ds(row0 + tok * D + j * 16, 16)]) * wchunk
                    for tok, acc in enumerate(accs))

            accs = lax.fori_loop(0, NL, _jstep, (zero,) * 16)

            packed = zero
            for tok, a in enumerate(accs):
                s = a
                for idx in rot_idx:
                    s = s + _rot(s, idx)
                packed = jnp.where(lanes == tok, s, packed)

            tv = packed + bvec
            outv[pl.ds(t * TILE_T + g * 16, 16)] = 1.0 / (1.0 + jnp.exp(-tv))

    pltpu.sync_copy(outv, out_hbm.at[batch, pl.ds(chunk * TOK_PER_W, TOK_PER_W)])


def _mask_body(wtc_ref, wsc_ref, weights_ref, mask_ref):
    weights_ref[:, :TC_S] = wtc_ref[...]
    weights_ref[:, TC_S:] = wsc_ref[...]
    u = lax.bitcast_convert_type(weights_ref[...], jnp.int32)
    lo0 = jnp.zeros((B, 1), jnp.int32)
    hi0 = jnp.full((B, 1), ONE_BITS, jnp.int32)

    def step(_, carry):
        lo, hi = carry
        mid = (lo + hi + 1) >> 1
        cnt = jnp.sum((u >= mid).astype(jnp.int32), axis=1, keepdims=True)
        ge = cnt >= K
        return jnp.where(ge, mid, lo), jnp.where(ge, hi, mid - 1)

    lo, _ = lax.fori_loop(0, 31, step, (lo0, hi0))
    mask_ref[...] = (u >= lo).astype(jnp.int32)


def kernel(hidden_states, W, b):
    weights_sc = _sc_weights(hidden_states.reshape(B, S * D), W.reshape(D),
                             jnp.broadcast_to(b, (16,)))
    weights_tc = _tc_weights(hidden_states, W, b)
    weights, mask = pl.pallas_call(
        _mask_body,
        out_shape=[
            jax.ShapeDtypeStruct((B, S), jnp.float32),
            jax.ShapeDtypeStruct((B, S), jnp.int32),
        ],
    )(weights_tc, weights_sc)
    return weights, mask.astype(bool)


# speedup vs baseline: 2.2183x; 2.2183x over previous
import functools

import jax
import jax.numpy as jnp
from jax import lax
from jax.experimental import pallas as pl
from jax.experimental.pallas import tpu as pltpu
from jax.experimental.pallas import tpu_sc as plsc

B, S, D = 4, 8192, 1024
K = max(1, int(0.5 * S))
ONE_BITS = 0x3F800000

TC_S = 4608
SC_S = S - TC_S
BLK = 512
NBLK = TC_S // BLK

NW = 32
TOK_PER_W = B * SC_S // NW
CHUNKS_PER_B = NW // B
TILE_T = 32
NT = TOK_PER_W // TILE_T
NL = D // 16

_SPLIT_C = 65537.0


def _bf16_round(x):
    t = x * _SPLIT_C
    return t - (t - x)


def _tc_matvec_body(hs_ref, w_ref, b_ref, weights_ref):
    i = pl.program_id(0)
    hs = hs_ref[...]
    w = w_ref[...]
    logits = jnp.dot(hs.reshape(B * BLK, D), w.reshape(D, 1),
                     preferred_element_type=jnp.float32)
    logits = logits.reshape(B, BLK) + b_ref[0, 0]
    weights_ref[...] = jax.nn.sigmoid(logits)


def _tc_weights(hidden_states, W, b):
    return pl.pallas_call(
        _tc_matvec_body,
        grid=(NBLK,),
        in_specs=[
            pl.BlockSpec((B, BLK, D), lambda i: (0, i, 0)),
            pl.BlockSpec((1, D), lambda i: (0, 0)),
            pl.BlockSpec((1, 1), lambda i: (0, 0)),
        ],
        out_specs=pl.BlockSpec((B, BLK), lambda i: (0, i)),
        out_shape=jax.ShapeDtypeStruct((B, TC_S), jnp.float32),
    )(hidden_states, W, b.reshape(1, 1))


@functools.partial(
    pl.kernel,
    mesh=plsc.VectorSubcoreMesh(core_axis_name="c", subcore_axis_name="s"),
    out_type=jax.ShapeDtypeStruct((B * SC_S,), jnp.float32),
    scratch_types=[
        pltpu.VMEM((D,), jnp.float32),
        pltpu.VMEM((16,), jnp.float32),
        pltpu.VMEM((TILE_T, D), jnp.float32),
        pltpu.VMEM((TILE_T, D), jnp.float32),
        pltpu.VMEM((TOK_PER_W,), jnp.float32),
        pltpu.SemaphoreType.DMA,
        pltpu.SemaphoreType.DMA,
    ],
)
def _sc_weights(hs_hbm, w_hbm, b_hbm, out_hbm,
                wv, bv, buf0, buf1, outv, sem0, sem1):
    cid = lax.axis_index("c")
    sid = lax.axis_index("s")
    wid = sid * 2 + cid
    batch = wid // CHUNKS_PER_B
    chunk = wid % CHUNKS_PER_B
    base = TC_S + chunk * TOK_PER_W

    pltpu.sync_copy(w_hbm.at[0], wv)
    pltpu.sync_copy(b_hbm, bv)
    bufs = (buf0, buf1)
    sems = (sem0, sem1)

    lanes = lax.iota(jnp.int32, 16)
    rot_idx = [(lanes + sh) & 15 for sh in (8, 4, 2, 1)]
    dn = lax.GatherDimensionNumbers(
        offset_dims=(), collapsed_slice_dims=(0,), start_index_map=(0,))

    def _rot(v, idx):
        return lax.gather(v, idx[:, None], dn, (1,),
                          mode=lax.GatherScatterMode.PROMISE_IN_BOUNDS)

    zero = jnp.zeros((16,), jnp.float32)
    bvec = bv[...]

    copies = [None, None]
    copies[0] = pltpu.async_copy(
        hs_hbm.at[batch, pl.ds(base, TILE_T), :], buf0, sems[0])
    for t in range(NT):
        if t + 1 < NT:
            copies[(t + 1) % 2] = pltpu.async_copy(
                hs_hbm.at[batch, pl.ds(base + (t + 1) * TILE_T, TILE_T), :],
                bufs[(t + 1) % 2], sems[(t + 1) % 2])
        copies[t % 2].wait()
        btile = bufs[t % 2]

        for g in range(TILE_T // 16):

            def _jstep(j, accs):
                wchunk = _bf16_round(wv[pl.ds(j * 16, 16)])
                return tuple(
                    acc + _bf16_round(
                        btile[g * 16 + tok, pl.ds(j * 16, 16)]) * wchunk
                    for tok, acc in enumerate(accs))

            accs = lax.fori_loop(0, NL, _jstep, (zero,) * 16)

            packed = zero
            for tok, a in enumerate(accs):
                s = a
                for idx in rot_idx:
                    s = s + _rot(s, idx)
                packed = jnp.where(lanes == tok, s, packed)

            tv = packed + bvec
            outv[pl.ds(t * TILE_T + g * 16, 16)] = 1.0 / (1.0 + jnp.exp(-tv))

    pltpu.sync_copy(outv, out_hbm.at[pl.ds(batch * SC_S + chunk * TOK_PER_W,
                                           TOK_PER_W)])


def _mask_body(wtc_ref, wsc_ref, weights_ref, mask_ref):
    weights_ref[:, :TC_S] = wtc_ref[...]
    weights_ref[:, TC_S:] = wsc_ref[...]
    u = lax.bitcast_convert_type(weights_ref[...], jnp.int32)
    lo0 = jnp.zeros((B, 1), jnp.int32)
    hi0 = jnp.full((B, 1), ONE_BITS, jnp.int32)

    def step(_, carry):
        lo, hi = carry
        mid = (lo + hi + 1) >> 1
        cnt = jnp.sum((u >= mid).astype(jnp.int32), axis=1, keepdims=True)
        ge = cnt >= K
        return jnp.where(ge, mid, lo), jnp.where(ge, hi, mid - 1)

    lo, _ = lax.fori_loop(0, 31, step, (lo0, hi0))
    mask_ref[...] = (u >= lo).astype(jnp.int32)


def kernel(hidden_states, W, b):
    weights_sc = _sc_weights(hidden_states, W,
                             jnp.broadcast_to(b, (16,))).reshape(B, SC_S)
    weights_tc = _tc_weights(hidden_states, W, b)
    weights, mask = pl.pallas_call(
        _mask_body,
        out_shape=[
            jax.ShapeDtypeStruct((B, S), jnp.float32),
            jax.ShapeDtypeStruct((B, S), jnp.int32),
        ],
    )(weights_tc, weights_sc)
    return weights, mask.astype(bool)


# speedup vs baseline: 2.4035x; 1.0835x over previous
import functools

import jax
import jax.numpy as jnp
from jax import lax
from jax.experimental import pallas as pl
from jax.experimental.pallas import tpu as pltpu
from jax.experimental.pallas import tpu_sc as plsc

B, S, D = 4, 8192, 1024
K = max(1, int(0.5 * S))
ONE_BITS = 0x3F800000

TC_S = 5632
SC_S = S - TC_S
BLK = 512
NBLK = TC_S // BLK

NW = 32
TOK_PER_W = B * SC_S // NW
CHUNKS_PER_B = NW // B
TILE_T = 32
NT = TOK_PER_W // TILE_T
NL = D // 16

_SPLIT_C = 65537.0


def _bf16_round(x):
    t = x * _SPLIT_C
    return t - (t - x)


def _tc_matvec_body(hs_ref, w_ref, b_ref, weights_ref):
    i = pl.program_id(0)
    hs = hs_ref[...]
    w = w_ref[...]
    logits = jnp.dot(hs.reshape(B * BLK, D), w.reshape(D, 1),
                     preferred_element_type=jnp.float32)
    logits = logits.reshape(B, BLK) + b_ref[0, 0]
    weights_ref[...] = jax.nn.sigmoid(logits)


def _tc_weights(hidden_states, W, b):
    return pl.pallas_call(
        _tc_matvec_body,
        grid=(NBLK,),
        in_specs=[
            pl.BlockSpec((B, BLK, D), lambda i: (0, i, 0)),
            pl.BlockSpec((1, D), lambda i: (0, 0)),
            pl.BlockSpec((1, 1), lambda i: (0, 0)),
        ],
        out_specs=pl.BlockSpec((B, BLK), lambda i: (0, i)),
        out_shape=jax.ShapeDtypeStruct((B, TC_S), jnp.float32),
    )(hidden_states, W, b.reshape(1, 1))


@functools.partial(
    pl.kernel,
    mesh=plsc.VectorSubcoreMesh(core_axis_name="c", subcore_axis_name="s"),
    out_type=jax.ShapeDtypeStruct((B * SC_S,), jnp.float32),
    scratch_types=[
        pltpu.VMEM((D,), jnp.float32),
        pltpu.VMEM((16,), jnp.float32),
        pltpu.VMEM((TILE_T, D), jnp.float32),
        pltpu.VMEM((TILE_T, D), jnp.float32),
        pltpu.VMEM((TOK_PER_W,), jnp.float32),
        pltpu.SemaphoreType.DMA,
        pltpu.SemaphoreType.DMA,
    ],
)
def _sc_weights(hs_hbm, w_hbm, b_hbm, out_hbm,
                wv, bv, buf0, buf1, outv, sem0, sem1):
    cid = lax.axis_index("c")
    sid = lax.axis_index("s")
    wid = sid * 2 + cid
    batch = wid // CHUNKS_PER_B
    chunk = wid % CHUNKS_PER_B
    base = TC_S + chunk * TOK_PER_W

    pltpu.sync_copy(w_hbm.at[0], wv)
    pltpu.sync_copy(b_hbm, bv)
    bufs = (buf0, buf1)
    sems = (sem0, sem1)

    lanes = lax.iota(jnp.int32, 16)
    rot_idx = [(lanes + sh) & 15 for sh in (8, 4, 2, 1)]
    dn = lax.GatherDimensionNumbers(
        offset_dims=(), collapsed_slice_dims=(0,), start_index_map=(0,))

    def _rot(v, idx):
        return lax.gather(v, idx[:, None], dn, (1,),
                          mode=lax.GatherScatterMode.PROMISE_IN_BOUNDS)

    zero = jnp.zeros((16,), jnp.float32)
    bvec = bv[...]

    copies = [None, None]
    copies[0] = pltpu.async_copy(
        hs_hbm.at[batch, pl.ds(base, TILE_T), :], buf0, sems[0])
    for t in range(NT):
        if t + 1 < NT:
            copies[(t + 1) % 2] = pltpu.async_copy(
                hs_hbm.at[batch, pl.ds(base + (t + 1) * TILE_T, TILE_T), :],
                bufs[(t + 1) % 2], sems[(t + 1) % 2])
        copies[t % 2].wait()
        btile = bufs[t % 2]

        for g in range(TILE_T // 16):

            def _jstep(j, accs):
                wchunk = _bf16_round(wv[pl.ds(j * 16, 16)])
                return tuple(
                    acc + _bf16_round(
                        btile[g * 16 + tok, pl.ds(j * 16, 16)]) * wchunk
                    for tok, acc in enumerate(accs))

            accs = lax.fori_loop(0, NL, _jstep, (zero,) * 16, unroll=2)

            packed = zero
            for tok, a in enumerate(accs):
                s = a
                for idx in rot_idx:
                    s = s + _rot(s, idx)
                packed = jnp.where(lanes == tok, s, packed)

            tv = packed + bvec
            outv[pl.ds(t * TILE_T + g * 16, 16)] = 1.0 / (1.0 + jnp.exp(-tv))

    pltpu.sync_copy(outv, out_hbm.at[pl.ds(batch * SC_S + chunk * TOK_PER_W,
                                           TOK_PER_W)])


def _mask_body(wtc_ref, wsc_ref, weights_ref, mask_ref):
    weights_ref[:, :TC_S] = wtc_ref[...]
    weights_ref[:, TC_S:] = wsc_ref[...]
    u = lax.bitcast_convert_type(weights_ref[...], jnp.int32)
    lo0 = jnp.zeros((B, 1), jnp.int32)
    hi0 = jnp.full((B, 1), ONE_BITS, jnp.int32)

    def step(_, carry):
        lo, hi = carry
        mid = (lo + hi + 1) >> 1
        cnt = jnp.sum((u >= mid).astype(jnp.int32), axis=1, keepdims=True)
        ge = cnt >= K
        return jnp.where(ge, mid, lo), jnp.where(ge, hi, mid - 1)

    lo, _ = lax.fori_loop(0, 31, step, (lo0, hi0))
    mask_ref[...] = (u >= lo).astype(jnp.int32)


def kernel(hidden_states, W, b):
    weights_sc = _sc_weights(hidden_states, W,
                             jnp.broadcast_to(b, (16,))).reshape(B, SC_S)
    weights_tc = _tc_weights(hidden_states, W, b)
    weights, mask = pl.pallas_call(
        _mask_body,
        out_shape=[
            jax.ShapeDtypeStruct((B, S), jnp.float32),
            jax.ShapeDtypeStruct((B, S), jnp.int32),
        ],
    )(weights_tc, weights_sc)
    return weights, mask.astype(bool)


# speedup vs baseline: 3.6198x; 1.5061x over previous
import jax
import jax.numpy as jnp
from jax import lax
from jax.experimental import pallas as pl

B, S, D = 4, 8192, 1024
BLK = 512
NBLK = S // BLK
K = max(1, int(0.5 * S))
ONE_BITS = 0x3F800000


def _body(hs_ref, w_ref, b_ref, weights_ref, mask_ref):
    i = pl.program_id(0)
    hs = hs_ref[...]
    w = w_ref[...]
    logits = jnp.dot(hs.reshape(B * BLK, D), w.reshape(D, 1),
                     preferred_element_type=jnp.float32)
    logits = logits.reshape(B, BLK) + b_ref[0, 0]
    weights_ref[:, pl.ds(i * BLK, BLK)] = jax.nn.sigmoid(logits)

    @pl.when(i == NBLK - 1)
    def _():
        u = lax.bitcast_convert_type(weights_ref[...], jnp.int32)
        lo0 = jnp.zeros((B, 1), jnp.int32)
        hi0 = jnp.full((B, 1), ONE_BITS, jnp.int32)

        def step(_, carry):
            lo, hi = carry
            mid = (lo + hi + 1) >> 1
            cnt = jnp.sum((u >= mid).astype(jnp.int32), axis=1, keepdims=True)
            ge = cnt >= K
            return jnp.where(ge, mid, lo), jnp.where(ge, hi, mid - 1)

        lo, _ = lax.fori_loop(0, 31, step, (lo0, hi0))
        mask_ref[...] = (u >= lo).astype(jnp.int32)


def kernel(hidden_states, W, b):
    weights, mask = pl.pallas_call(
        _body,
        grid=(NBLK,),
        in_specs=[
            pl.BlockSpec((B, BLK, D), lambda i: (0, i, 0)),
            pl.BlockSpec((1, D), lambda i: (0, 0)),
            pl.BlockSpec((1, 1), lambda i: (0, 0)),
        ],
        out_specs=[
            pl.BlockSpec((B, S), lambda i: (0, 0)),
            pl.BlockSpec((B, S), lambda i: (0, 0)),
        ],
        out_shape=[
            jax.ShapeDtypeStruct((B, S), jnp.float32),
            jax.ShapeDtypeStruct((B, S), jnp.int32),
        ],
    )(hidden_states, W, b.reshape(1, 1))
    return weights, mask.astype(bool)
